# baseline (speedup 1.0000x reference)
import functools

import jax
import jax.numpy as jnp
from jax import lax
from jax.experimental import pallas as pl
from jax.experimental.pallas import tpu as pltpu
from jax.experimental.pallas import tpu_sc as plsc

B = 4
S = 2048
D = 2048
NW = 32
ROWS = S * B
RPW = ROWS // NW
K = 16
NCH = RPW // K
WPC = K // B
LANES = 16

_mesh = plsc.VectorSubcoreMesh(core_axis_name="c", subcore_axis_name="s")


@functools.partial(
    pl.kernel,
    mesh=_mesh,
    out_type=jax.ShapeDtypeStruct((ROWS, D), jnp.float32),
    scratch_types=[
        pltpu.VMEM((NCH, K), jnp.int32),
        pltpu.VMEM((K, D), jnp.float32),
        pltpu.VMEM((WPC, D), jnp.float32),
        pltpu.SemaphoreType.DMA,
    ],
)
def _embed(idx_hbm, wte_hbm, wpe_hbm, out_hbm, idx_v, buf, wpe_v, sem):
    wid = lax.axis_index("s") * 2 + lax.axis_index("c")
    pltpu.sync_copy(idx_hbm.at[wid], idx_v)

    def chunk(c, carry):
        row0 = wid * RPW + c * K
        gather = pltpu.async_copy(wte_hbm.at[idx_v.at[c]], buf, sem)
        pltpu.sync_copy(
            wpe_hbm.at[pl.ds(wid * (RPW // B) + c * WPC, WPC)], wpe_v
        )
        gather.wait()

        def add_t(t, acc):
            col = pl.ds(t * LANES, LANES)
            for q in range(WPC):
                w = wpe_v[q, col]
                for j in range(B):
                    plsc.addupdate(buf.at[B * q + j, col], w)
            return acc

        lax.fori_loop(0, D // LANES, add_t, 0)
        pltpu.sync_copy(buf, out_hbm.at[pl.ds(row0, K)])
        return carry

    lax.fori_loop(0, NCH, chunk, 0)


def kernel(input_ids, wte, wpe):
    tokens = input_ids[:, :-1]
    labels = input_ids[:, 1:]
    idx = tokens.astype(jnp.int32).T.reshape(NW, NCH, K)
    hs = _embed(idx, wte, wpe)
    return (hs.reshape(S, B, D), labels)


# speedup vs baseline: 1.2188x; 1.2188x over previous
import functools

import jax
import jax.numpy as jnp
from jax import lax
from jax.experimental import pallas as pl
from jax.experimental.pallas import tpu as pltpu
from jax.experimental.pallas import tpu_sc as plsc

B = 4
S = 2048
D = 2048
NW = 32
ROWS = S * B
RPW = ROWS // NW
K = 16
NCH = RPW // K
WPC = K // B
LANES = 16
NBUF = 3

_mesh = plsc.VectorSubcoreMesh(core_axis_name="c", subcore_axis_name="s")


@functools.partial(
    pl.kernel,
    mesh=_mesh,
    out_type=jax.ShapeDtypeStruct((ROWS, D), jnp.float32),
    scratch_types=[
        pltpu.VMEM((NCH, K), jnp.int32),
        pltpu.VMEM((NBUF, K, D), jnp.float32),
        pltpu.VMEM((NBUF, WPC, D), jnp.float32),
        pltpu.SemaphoreType.DMA((NBUF,)),
        pltpu.SemaphoreType.DMA((NBUF,)),
        pltpu.SemaphoreType.DMA((NBUF,)),
    ],
)
def _embed(idx_hbm, wte_hbm, wpe_hbm, out_hbm, idx_v, bufs, wpes,
           gsem, wsem, osem):
    wid = lax.axis_index("s") * 2 + lax.axis_index("c")
    pltpu.sync_copy(idx_hbm.at[wid], idx_v)

    def issue(c):
        s = c % NBUF
        g = pltpu.async_copy(wte_hbm.at[idx_v.at[c]], bufs.at[s],
                             gsem.at[s])
        w = pltpu.async_copy(
            wpe_hbm.at[pl.ds(wid * (RPW // B) + c * WPC, WPC)],
            wpes.at[s], wsem.at[s])
        return g, w

    gw = {c: issue(c) for c in range(NBUF)}
    out_descs = {}
    for c in range(NCH):
        s = c % NBUF
        if c >= 1 and c + NBUF - 1 < NCH:
            out_descs[c - 1].wait()
            gw[c + NBUF - 1] = issue(c + NBUF - 1)
        g, w = gw.pop(c)
        g.wait()
        w.wait()

        def add_t(t, acc):
            col = pl.ds(t * LANES, LANES)
            for q in range(WPC):
                wv = wpes[s, q, col]
                for j in range(B):
                    plsc.addupdate(bufs.at[s, B * q + j, col], wv)
            return acc

        lax.fori_loop(0, D // LANES, add_t, 0)
        out_descs[c] = pltpu.async_copy(
            bufs.at[s], out_hbm.at[pl.ds(wid * RPW + c * K, K)],
            osem.at[s])
    for c in range(NCH - NBUF, NCH):
        out_descs[c].wait()


def kernel(input_ids, wte, wpe):
    tokens = input_ids[:, :-1]
    labels = input_ids[:, 1:]
    idx = tokens.astype(jnp.int32).T.reshape(NW, NCH, K)
    hs = _embed(idx, wte, wpe)
    return (hs.reshape(S, B, D), labels)


# speedup vs baseline: 2.0500x; 1.6819x over previous
import functools

import jax
import jax.numpy as jnp
from jax import lax
from jax.experimental import pallas as pl
from jax.experimental.pallas import tpu as pltpu
from jax.experimental.pallas import tpu_sc as plsc

B = 4
S = 2048
D = 2048
NW = 32
ROWS = S * B
RPW = ROWS // NW
K = 16
NCH = RPW // K
WPC = K // B
LANES = 16
NBUF = 3

_mesh = plsc.VectorSubcoreMesh(core_axis_name="c", subcore_axis_name="s")


@functools.partial(
    pl.kernel,
    mesh=_mesh,
    out_type=jax.ShapeDtypeStruct((S, B, D), jnp.float32),
    scratch_types=[
        pltpu.VMEM((NCH, K), jnp.int32),
        pltpu.VMEM((NBUF, K, D), jnp.float32),
        pltpu.VMEM((NBUF, WPC, D), jnp.float32),
        pltpu.SemaphoreType.DMA((NBUF,)),
        pltpu.SemaphoreType.DMA((NBUF,)),
        pltpu.SemaphoreType.DMA((NBUF,)),
    ],
)
def _embed(idx_hbm, wte_hbm, wpe_hbm, out_hbm, idx_v, bufs, wpes,
           gsem, wsem, osem):
    wid = lax.axis_index("s") * 2 + lax.axis_index("c")
    pltpu.sync_copy(idx_hbm.at[wid], idx_v)

    def issue(c):
        s = c % NBUF
        g = pltpu.async_copy(wte_hbm.at[idx_v.at[c]], bufs.at[s],
                             gsem.at[s])
        w = pltpu.async_copy(
            wpe_hbm.at[pl.ds(wid * (RPW // B) + c * WPC, WPC)],
            wpes.at[s], wsem.at[s])
        return g, w

    gw = {c: issue(c) for c in range(NBUF)}
    out_descs = {}
    for c in range(NCH):
        s = c % NBUF
        if c >= 1 and c + NBUF - 1 < NCH:
            for d in out_descs[c - 1]:
                d.wait()
            gw[c + NBUF - 1] = issue(c + NBUF - 1)
        g, w = gw.pop(c)
        g.wait()
        w.wait()

        def add_t(t, acc):
            col = pl.ds(t * LANES, LANES)
            for q in range(WPC):
                wv = wpes[s, q, col]
                for j in range(B):
                    plsc.addupdate(bufs.at[s, B * q + j, col], wv)
            return acc

        lax.fori_loop(0, D // LANES, add_t, 0)
        s0 = wid * (RPW // B) + c * WPC
        out_descs[c] = [
            pltpu.async_copy(bufs.at[s, pl.ds(B * q, B)],
                             out_hbm.at[s0 + q], osem.at[s])
            for q in range(WPC)
        ]
    for c in range(NCH - NBUF, NCH):
        for d in out_descs[c]:
            d.wait()


def kernel(input_ids, wte, wpe):
    tokens = input_ids[:, :-1]
    labels = input_ids[:, 1:]
    idx = tokens.astype(jnp.int32).T.reshape(NW, NCH, K)
    hs = _embed(idx, wte, wpe)
    return (hs, labels)


# speedup vs baseline: 2.2844x; 1.1143x over previous
import functools

import jax
import jax.numpy as jnp
from jax import lax
from jax.experimental import pallas as pl
from jax.experimental.pallas import tpu as pltpu
from jax.experimental.pallas import tpu_sc as plsc

B = 4
S = 2048
D = 2048
NW = 32
ROWS = S * B
RPW = ROWS // NW
K = 16
NCH = RPW // K
WPC = K // B
LANES = 16
NBUF = 3

_mesh = plsc.VectorSubcoreMesh(core_axis_name="c", subcore_axis_name="s")


@functools.partial(
    pl.kernel,
    mesh=_mesh,
    out_type=jax.ShapeDtypeStruct((S, B, D), jnp.float32),
    scratch_types=[
        pltpu.VMEM((NCH, K), jnp.int32),
        pltpu.VMEM((NBUF, K, D), jnp.float32),
        pltpu.VMEM((NBUF, WPC, D), jnp.float32),
        pltpu.SemaphoreType.DMA((NBUF,)),
        pltpu.SemaphoreType.DMA((NBUF,)),
        pltpu.SemaphoreType.DMA((NBUF,)),
    ],
)
def _embed(idx_hbm, wte_hbm, wpe_hbm, out_hbm, idx_v, bufs, wpes,
           gsem, wsem, osem):
    wid = lax.axis_index("s") * 2 + lax.axis_index("c")
    pltpu.sync_copy(idx_hbm.at[wid], idx_v)

    def issue(c):
        s = c % NBUF
        g = pltpu.async_copy(wte_hbm.at[idx_v.at[c]], bufs.at[s],
                             gsem.at[s])
        w = pltpu.async_copy(
            wpe_hbm.at[pl.ds(wid * (RPW // B) + c * WPC, WPC)],
            wpes.at[s], wsem.at[s])
        return g, w

    gw = {c: issue(c) for c in range(NBUF)}
    out_descs = {}
    for c in range(NCH):
        s = c % NBUF
        if c >= 1 and c + NBUF - 1 < NCH:
            for d in out_descs[c - 1]:
                d.wait()
            gw[c + NBUF - 1] = issue(c + NBUF - 1)
        g, w = gw.pop(c)
        g.wait()
        w.wait()
        s0 = wid * (RPW // B) + c * WPC
        out_descs[c] = []
        for q in range(WPC):
            def add_t(t, acc, q=q):
                c0 = pl.ds(t * 2 * LANES, LANES)
                c1 = pl.ds(t * 2 * LANES + LANES, LANES)
                w0 = wpes[s, q, c0]
                w1 = wpes[s, q, c1]
                for j in range(B):
                    plsc.addupdate(bufs.at[s, B * q + j, c0], w0)
                    plsc.addupdate(bufs.at[s, B * q + j, c1], w1)
                return acc

            lax.fori_loop(0, D // (2 * LANES), add_t, 0)
            out_descs[c].append(
                pltpu.async_copy(bufs.at[s, pl.ds(B * q, B)],
                                 out_hbm.at[s0 + q], osem.at[s]))
    for c in range(NCH - NBUF, NCH):
        for d in out_descs[c]:
            d.wait()


def kernel(input_ids, wte, wpe):
    tokens = input_ids[:, :-1]
    labels = input_ids[:, 1:]
    idx = tokens.astype(jnp.int32).T.reshape(NW, NCH, K)
    hs = _embed(idx, wte, wpe)
    return (hs, labels)


# speedup vs baseline: 2.3253x; 1.0179x over previous
import functools

import jax
import jax.numpy as jnp
from jax import lax
from jax.experimental import pallas as pl
from jax.experimental.pallas import tpu as pltpu
from jax.experimental.pallas import tpu_sc as plsc

B = 4
S = 2048
D = 2048
NW = 32
ROWS = S * B
RPW = ROWS // NW
K = 8
NCH = RPW // K
WPC = K // B
LANES = 16
NBUF = 6
PREF = 4

_mesh = plsc.VectorSubcoreMesh(core_axis_name="c", subcore_axis_name="s")


@functools.partial(
    pl.kernel,
    mesh=_mesh,
    out_type=jax.ShapeDtypeStruct((S, B, D), jnp.float32),
    scratch_types=[
        pltpu.VMEM((NCH, K), jnp.int32),
        pltpu.VMEM((NBUF, K, D), jnp.float32),
        pltpu.VMEM((NBUF, WPC, D), jnp.float32),
        pltpu.SemaphoreType.DMA((NBUF,)),
        pltpu.SemaphoreType.DMA((NBUF,)),
        pltpu.SemaphoreType.DMA((NBUF,)),
    ],
)
def _embed(idx_hbm, wte_hbm, wpe_hbm, out_hbm, idx_v, bufs, wpes,
           gsem, wsem, osem):
    wid = lax.axis_index("s") * 2 + lax.axis_index("c")
    pltpu.sync_copy(idx_hbm.at[wid], idx_v)

    def issue(c):
        s = c % NBUF
        g = pltpu.async_copy(wte_hbm.at[idx_v.at[c]], bufs.at[s],
                             gsem.at[s])
        w = pltpu.async_copy(
            wpe_hbm.at[pl.ds(wid * (RPW // B) + c * WPC, WPC)],
            wpes.at[s], wsem.at[s])
        return g, w

    gw = {c: issue(c) for c in range(PREF)}
    out_descs = {}
    for c in range(NCH):
        s = c % NBUF
        cn = c + PREF
        if cn < NCH:
            if cn - NBUF >= 0:
                for d in out_descs[cn - NBUF]:
                    d.wait()
            gw[cn] = issue(cn)
        g, w = gw.pop(c)
        g.wait()
        w.wait()
        s0 = wid * (RPW // B) + c * WPC
        out_descs[c] = []
        for q in range(WPC):
            def add_t(t, acc, q=q):
                c0 = pl.ds(t * 2 * LANES, LANES)
                c1 = pl.ds(t * 2 * LANES + LANES, LANES)
                w0 = wpes[s, q, c0]
                w1 = wpes[s, q, c1]
                for j in range(B):
                    plsc.addupdate(bufs.at[s, B * q + j, c0], w0)
                    plsc.addupdate(bufs.at[s, B * q + j, c1], w1)
                return acc

            lax.fori_loop(0, D // (2 * LANES), add_t, 0)
            out_descs[c].append(
                pltpu.async_copy(bufs.at[s, pl.ds(B * q, B)],
                                 out_hbm.at[s0 + q], osem.at[s]))
    for c in range(NCH - NBUF, NCH):
        for d in out_descs[c]:
            d.wait()


def kernel(input_ids, wte, wpe):
    tokens = input_ids[:, :-1]
    labels = input_ids[:, 1:]
    idx = tokens.astype(jnp.int32).T.reshape(NW, NCH, K)
    hs = _embed(idx, wte, wpe)
    return (hs, labels)


# speedup vs baseline: 2.3441x; 1.0081x over previous
import functools

import jax
import jax.numpy as jnp
from jax import lax
from jax.experimental import pallas as pl
from jax.experimental.pallas import tpu as pltpu
from jax.experimental.pallas import tpu_sc as plsc

B = 4
S = 2048
D = 2048
NW = 32
ROWS = S * B
RPW = ROWS // NW
K = 8
NCH = RPW // K
WPC = K // B
LANES = 16
NBUF = 6
PREF = 4

_mesh = plsc.VectorSubcoreMesh(core_axis_name="c", subcore_axis_name="s")


@functools.partial(
    pl.kernel,
    mesh=_mesh,
    out_type=jax.ShapeDtypeStruct((S, B, D), jnp.float32),
    scratch_types=[
        pltpu.VMEM((NCH, K), jnp.int32),
        pltpu.VMEM((NBUF, K, D), jnp.float32),
        pltpu.VMEM((NBUF, WPC, D), jnp.float32),
        pltpu.SemaphoreType.DMA((NBUF,)),
        pltpu.SemaphoreType.DMA((NBUF,)),
        pltpu.SemaphoreType.DMA((NBUF,)),
    ],
)
def _embed(idx_hbm, wte_hbm, wpe_hbm, out_hbm, idx_v, bufs, wpes,
           gsem, wsem, osem):
    wid = lax.axis_index("s") * 2 + lax.axis_index("c")
    pltpu.sync_copy(idx_hbm.at[wid], idx_v)

    def issue(c):
        s = c % NBUF
        g = pltpu.async_copy(wte_hbm.at[idx_v.at[c]], bufs.at[s],
                             gsem.at[s])
        w = pltpu.async_copy(
            wpe_hbm.at[pl.ds(wid * (RPW // B) + c * WPC, WPC)],
            wpes.at[s], wsem.at[s])
        return g, w

    gw = {c: issue(c) for c in range(PREF)}
    out_descs = {}
    for c in range(NCH):
        s = c % NBUF
        cn = c + PREF
        if cn < NCH:
            if cn - NBUF >= 0:
                for d in out_descs[cn - NBUF]:
                    d.wait()
            gw[cn] = issue(cn)
        g, w = gw.pop(c)
        g.wait()
        w.wait()
        s0 = wid * (RPW // B) + c * WPC

        @plsc.parallel_loop(0, D, step=2 * LANES, carry=jnp.int32(0))
        def _add(i, acc):
            c0 = pl.ds(i, LANES)
            c1 = pl.ds(i + LANES, LANES)
            for q in range(WPC):
                w0 = wpes[s, q, c0]
                w1 = wpes[s, q, c1]
                for j in range(B):
                    plsc.addupdate(bufs.at[s, B * q + j, c0], w0)
                    plsc.addupdate(bufs.at[s, B * q + j, c1], w1)
            return acc

        out_descs[c] = [
            pltpu.async_copy(bufs.at[s, pl.ds(B * q, B)],
                             out_hbm.at[s0 + q], osem.at[s])
            for q in range(WPC)
        ]
    for c in range(NCH - NBUF, NCH):
        for d in out_descs[c]:
            d.wait()


def kernel(input_ids, wte, wpe):
    tokens = input_ids[:, :-1]
    labels = input_ids[:, 1:]
    idx = tokens.astype(jnp.int32).T.reshape(NW, NCH, K)
    hs = _embed(idx, wte, wpe)
    return (hs, labels)
